# baseline (speedup 1.0000x reference)
import functools

import jax
import jax.numpy as jnp
from jax import lax
from jax.experimental import pallas as pl
from jax.experimental.pallas import tpu as pltpu
from jax.experimental.pallas import tpu_sc as plsc

_B = 8
_NQ = 8
_L = 2048
_D = 256
_LC = 128
_NCH = _L // _LC
_NW = 32
_SPW = (_B * _NQ) // _NW


def _sc_body(x_hbm, emb_hbm, out_hbm, emb_v, buf_v):
    c = lax.axis_index("c")
    s = lax.axis_index("s")
    wid = s * 2 + c
    pltpu.sync_copy(emb_hbm, emb_v)
    for k in range(_SPW):
        sid = wid * _SPW + k
        b = sid // _NQ
        q = sid - b * _NQ
        e = [emb_v[q, pl.ds(j * 16, 16)] for j in range(16)]

        def chunk(i, carry):
            l0 = i * _LC
            pltpu.sync_copy(x_hbm.at[b, q, pl.ds(l0, _LC), :], buf_v)

            def row(l, rcarry):
                for j in range(16):
                    sl = pl.ds(j * 16, 16)
                    buf_v[l, sl] = buf_v[l, sl] + e[j]
                return rcarry

            lax.fori_loop(0, _LC, row, 0, unroll=2)
            pltpu.sync_copy(
                buf_v, out_hbm.at[b, pl.ds(l0, _LC), pl.ds(q * _D, _D)]
            )
            return carry

        lax.fori_loop(0, _NCH, chunk, 0)


@functools.partial(jax.jit, static_argnames=())
def _sc_call(x, quantizer_emb):
    mesh = plsc.VectorSubcoreMesh(core_axis_name="c", subcore_axis_name="s")
    f = pl.kernel(
        _sc_body,
        out_type=jax.ShapeDtypeStruct((_B, _L, _NQ * _D), jnp.float32),
        mesh=mesh,
        scratch_types=[
            pltpu.VMEM((_NQ, _D), jnp.float32),
            pltpu.VMEM((_LC, _D), jnp.float32),
        ],
    )
    return f(x, quantizer_emb)


def kernel(x, quantizer_emb):
    return _sc_call(x, quantizer_emb)


# speedup vs baseline: 2.0671x; 2.0671x over previous
import jax
import jax.numpy as jnp
from jax.experimental import pallas as pl

_NQ = 8
_D = 256
_LT = 1024


def _body(x_ref, emb_ref, o_ref):
    for qi in range(_NQ):
        o_ref[0, :, qi * _D:(qi + 1) * _D] = x_ref[0, qi] + emb_ref[qi]


def kernel(x, quantizer_emb):
    b, q, l, d = x.shape
    grid = (b, l // _LT)
    out = pl.pallas_call(
        _body,
        grid=grid,
        in_specs=[
            pl.BlockSpec((1, q, _LT, d), lambda i, j: (i, 0, j, 0)),
            pl.BlockSpec((q, d), lambda i, j: (0, 0)),
        ],
        out_specs=pl.BlockSpec((1, _LT, q * d), lambda i, j: (i, j, 0)),
        out_shape=jax.ShapeDtypeStruct((b, l, q * d), x.dtype),
    )(x, quantizer_emb)
    return out
